# baseline (speedup 1.0000x reference)
import functools

import jax
import jax.numpy as jnp
from jax import lax
from jax.experimental import pallas as pl
from jax.experimental.pallas import tpu as pltpu
from jax.experimental.pallas import tpu_sc as plsc

_GAMMA = 12.0
_B = 16384
_D = 128
_L = 16
_NC, _NS = 2, 16
_NW = _NC * _NS
_BPW = _B // _NW
_CHUNK = 64
_NCHUNK = _BPW // _CHUNK
_NSLOT = 4
_DV = _D // _L

_mesh = plsc.VectorSubcoreMesh(core_axis_name="c", subcore_axis_name="s")


@functools.partial(
    pl.kernel,
    out_type=jax.ShapeDtypeStruct((_B,), jnp.float32),
    mesh=_mesh,
    scratch_types=[
        pltpu.VMEM((_BPW,), jnp.int32),
        pltpu.VMEM((_BPW,), jnp.int32),
        pltpu.VMEM((_BPW,), jnp.int32),
        pltpu.VMEM((_NSLOT, _CHUNK, _D), jnp.float32),
        pltpu.VMEM((_NSLOT, _CHUNK, _D), jnp.float32),
        pltpu.VMEM((_NSLOT, _CHUNK, _D), jnp.float32),
        pltpu.VMEM((_BPW,), jnp.float32),
        pltpu.SemaphoreType.DMA,
        pltpu.SemaphoreType.DMA,
        pltpu.SemaphoreType.DMA,
        pltpu.SemaphoreType.DMA,
    ],
)
def _transe_sc(hi_hbm, ri_hbm, ti_hbm, ent_hbm, rel_hbm, out_hbm,
               hi_v, ri_v, ti_v, h_v, r_v, t_v, out_v,
               sem0, sem1, sem2, sem3):
    wid = lax.axis_index("s") * _NC + lax.axis_index("c")
    base = wid * _BPW

    pltpu.sync_copy(hi_hbm.at[pl.ds(base, _BPW)], hi_v)
    pltpu.sync_copy(ri_hbm.at[pl.ds(base, _BPW)], ri_v)
    pltpu.sync_copy(ti_hbm.at[pl.ds(base, _BPW)], ti_v)

    sems = (sem0, sem1, sem2, sem3)
    lanes = lax.iota(jnp.int32, _L)
    perm8 = jnp.bitwise_xor(lanes, 8)
    perm4 = jnp.bitwise_xor(lanes, 4)

    def start_gathers(c, slot):
        off = c * _CHUNK
        sem = sems[slot]
        d0 = pltpu.async_copy(ent_hbm.at[hi_v.at[pl.ds(off, _CHUNK)]],
                              h_v.at[slot], sem)
        d1 = pltpu.async_copy(rel_hbm.at[ri_v.at[pl.ds(off, _CHUNK)]],
                              r_v.at[slot], sem)
        d2 = pltpu.async_copy(ent_hbm.at[ti_v.at[pl.ds(off, _CHUNK)]],
                              t_v.at[slot], sem)
        return (d0, d1, d2)

    def compute_chunk(c, slot):
        hs, rs, ts = h_v.at[slot], r_v.at[slot], t_v.at[slot]
        out_off = c * _CHUNK

        @plsc.parallel_loop(0, _CHUNK // _L)
        def body(g):
            i0 = g * _L
            score = jnp.full((_L,), _GAMMA, jnp.float32)
            for k in range(_L):
                acc = jnp.zeros((_L,), jnp.float32)
                for j in range(_DV):
                    dsl = pl.ds(j * _L, _L)
                    acc = acc + jnp.abs(
                        hs[i0 + k, dsl] + rs[i0 + k, dsl] - ts[i0 + k, dsl])
                acc = acc + jnp.take(acc, perm8)
                acc = acc + jnp.take(acc, perm4)
                s = ((acc[0] + acc[1]) + (acc[2] + acc[3]))
                score = score - jnp.where(lanes == k, s, 0.0)
            out_v[pl.ds(out_off + i0, _L)] = score

    depth = _NSLOT - 1
    pending = [start_gathers(c, c % _NSLOT) for c in range(depth)]
    for c in range(_NCHUNK):
        for d in pending.pop(0):
            d.wait()
        if c + depth < _NCHUNK:
            pending.append(start_gathers(c + depth, (c + depth) % _NSLOT))
        compute_chunk(c, c % _NSLOT)

    pltpu.sync_copy(out_v, out_hbm.at[pl.ds(base, _BPW)])


def kernel(sample, entity_embedding, relation_embedding):
    hi = sample[:, 0]
    ri = sample[:, 1]
    ti = sample[:, 2]
    out = _transe_sc(hi, ri, ti, entity_embedding, relation_embedding)
    return out[:, None]


# speedup vs baseline: 1.1090x; 1.1090x over previous
import functools

import jax
import jax.numpy as jnp
from jax import lax
from jax.experimental import pallas as pl
from jax.experimental.pallas import tpu as pltpu
from jax.experimental.pallas import tpu_sc as plsc

_GAMMA = 12.0
_B = 16384
_D = 128
_L = 16
_NC, _NS = 2, 16
_NW = _NC * _NS
_BPW = _B // _NW
_CHUNK = 64
_NCHUNK = _BPW // _CHUNK
_NSLOT = 4
_DV = _D // _L

_mesh = plsc.VectorSubcoreMesh(core_axis_name="c", subcore_axis_name="s")


@functools.partial(
    pl.kernel,
    out_type=jax.ShapeDtypeStruct((_B,), jnp.float32),
    mesh=_mesh,
    scratch_types=[
        pltpu.VMEM((_BPW,), jnp.int32),
        pltpu.VMEM((_BPW,), jnp.int32),
        pltpu.VMEM((_BPW,), jnp.int32),
        pltpu.VMEM((_NSLOT, _CHUNK, _D), jnp.float32),
        pltpu.VMEM((_NSLOT, _CHUNK, _D), jnp.float32),
        pltpu.VMEM((_NSLOT, _CHUNK, _D), jnp.float32),
        pltpu.VMEM((_BPW,), jnp.float32),
        pltpu.SemaphoreType.DMA,
        pltpu.SemaphoreType.DMA,
        pltpu.SemaphoreType.DMA,
        pltpu.SemaphoreType.DMA,
    ],
)
def _transe_sc(hi_hbm, ri_hbm, ti_hbm, ent_hbm, rel_hbm, out_hbm,
               hi_v, ri_v, ti_v, h_v, r_v, t_v, out_v,
               sem0, sem1, sem2, sem3):
    wid = lax.axis_index("s") * _NC + lax.axis_index("c")
    base = wid * _BPW

    pltpu.sync_copy(hi_hbm.at[pl.ds(base, _BPW)], hi_v)
    pltpu.sync_copy(ri_hbm.at[pl.ds(base, _BPW)], ri_v)
    pltpu.sync_copy(ti_hbm.at[pl.ds(base, _BPW)], ti_v)

    sems = (sem0, sem1, sem2, sem3)
    lanes = lax.iota(jnp.int32, _L)

    def start_gathers(c, slot):
        off = c * _CHUNK
        sem = sems[slot]
        d0 = pltpu.async_copy(ent_hbm.at[hi_v.at[pl.ds(off, _CHUNK)]],
                              h_v.at[slot], sem)
        d1 = pltpu.async_copy(rel_hbm.at[ri_v.at[pl.ds(off, _CHUNK)]],
                              r_v.at[slot], sem)
        d2 = pltpu.async_copy(ent_hbm.at[ti_v.at[pl.ds(off, _CHUNK)]],
                              t_v.at[slot], sem)
        return (d0, d1, d2)

    def compute_chunk(c, slot):
        hs, rs, ts = h_v.at[slot], r_v.at[slot], t_v.at[slot]
        out_off = c * _CHUNK

        @plsc.parallel_loop(0, _CHUNK // _L)
        def body(g):
            i0 = g * _L
            score = jnp.full((_L,), _GAMMA, jnp.float32)
            for k in range(_L):
                acc = jnp.zeros((_L,), jnp.float32)
                for j in range(_DV):
                    dsl = pl.ds(j * _L, _L)
                    acc = acc + jnp.abs(
                        hs[i0 + k, dsl] + rs[i0 + k, dsl] - ts[i0 + k, dsl])
                e = [acc[m] for m in range(_L)]
                while len(e) > 1:
                    e = [a + b for a, b in zip(e[::2], e[1::2])]
                s = e[0]
                score = score - jnp.where(lanes == k, s, 0.0)
            out_v[pl.ds(out_off + i0, _L)] = score

    depth = _NSLOT - 1
    pending = [start_gathers(c, c % _NSLOT) for c in range(depth)]
    for c in range(_NCHUNK):
        for d in pending.pop(0):
            d.wait()
        if c + depth < _NCHUNK:
            pending.append(start_gathers(c + depth, (c + depth) % _NSLOT))
        compute_chunk(c, c % _NSLOT)

    pltpu.sync_copy(out_v, out_hbm.at[pl.ds(base, _BPW)])


def kernel(sample, entity_embedding, relation_embedding):
    hi = sample[:, 0]
    ri = sample[:, 1]
    ti = sample[:, 2]
    out = _transe_sc(hi, ri, ti, entity_embedding, relation_embedding)
    return out[:, None]


# speedup vs baseline: 1.2521x; 1.1290x over previous
import functools

import jax
import jax.numpy as jnp
from jax import lax
from jax.experimental import pallas as pl
from jax.experimental.pallas import tpu as pltpu
from jax.experimental.pallas import tpu_sc as plsc

_GAMMA = 12.0
_B = 16384
_D = 128
_L = 16
_NC, _NS = 2, 16
_NW = _NC * _NS
_BPW = _B // _NW
_CHUNK = 128
_NCHUNK = _BPW // _CHUNK
_DV = _D // _L

_mesh = plsc.VectorSubcoreMesh(core_axis_name="c", subcore_axis_name="s")


@functools.partial(
    pl.kernel,
    out_type=jax.ShapeDtypeStruct((_B,), jnp.float32),
    mesh=_mesh,
    scratch_types=[
        pltpu.VMEM((_BPW,), jnp.int32),
        pltpu.VMEM((_BPW,), jnp.int32),
        pltpu.VMEM((_BPW,), jnp.int32),
        pltpu.VMEM((2, _CHUNK, _D), jnp.float32),
        pltpu.VMEM((2, _CHUNK, _D), jnp.float32),
        pltpu.VMEM((2, _CHUNK, _D), jnp.float32),
        pltpu.VMEM((_BPW,), jnp.float32),
        pltpu.SemaphoreType.DMA,
        pltpu.SemaphoreType.DMA,
    ],
)
def _transe_sc(hi_hbm, ri_hbm, ti_hbm, ent_hbm, rel_hbm, out_hbm,
               hi_v, ri_v, ti_v, h_v, r_v, t_v, out_v, sem0, sem1):
    wid = lax.axis_index("s") * _NC + lax.axis_index("c")
    base = wid * _BPW

    pltpu.sync_copy(hi_hbm.at[pl.ds(base, _BPW)], hi_v)
    pltpu.sync_copy(ri_hbm.at[pl.ds(base, _BPW)], ri_v)
    pltpu.sync_copy(ti_hbm.at[pl.ds(base, _BPW)], ti_v)

    sems = (sem0, sem1)
    lanes = lax.iota(jnp.int32, _L)

    def start_gathers(c, slot):
        off = c * _CHUNK
        sem = sems[slot]
        d0 = pltpu.async_copy(ent_hbm.at[hi_v.at[pl.ds(off, _CHUNK)]],
                              h_v.at[slot], sem)
        d1 = pltpu.async_copy(rel_hbm.at[ri_v.at[pl.ds(off, _CHUNK)]],
                              r_v.at[slot], sem)
        d2 = pltpu.async_copy(ent_hbm.at[ti_v.at[pl.ds(off, _CHUNK)]],
                              t_v.at[slot], sem)
        return (d0, d1, d2)

    def compute_chunk(c, slot):
        hs, rs, ts = h_v.at[slot], r_v.at[slot], t_v.at[slot]
        out_off = c * _CHUNK

        def body(g, _):
            i0 = g * _L
            score = jnp.full((_L,), _GAMMA, jnp.float32)
            for k in range(_L):
                acc = jnp.zeros((_L,), jnp.float32)
                for j in range(_DV):
                    dsl = pl.ds(j * _L, _L)
                    acc = acc + jnp.abs(
                        hs[i0 + k, dsl] + rs[i0 + k, dsl] - ts[i0 + k, dsl])
                e = [acc[m] for m in range(_L)]
                while len(e) > 1:
                    e = [a + b for a, b in zip(e[::2], e[1::2])]
                score = score - jnp.where(lanes == k, e[0], 0.0)
            out_v[pl.ds(out_off + i0, _L)] = score
            return 0

        lax.fori_loop(0, _CHUNK // _L, body, 0)

    pending = start_gathers(0, 0)
    for c in range(_NCHUNK):
        for d in pending:
            d.wait()
        if c + 1 < _NCHUNK:
            pending = start_gathers(c + 1, (c + 1) % 2)
        compute_chunk(c, c % 2)

    pltpu.sync_copy(out_v, out_hbm.at[pl.ds(base, _BPW)])


def kernel(sample, entity_embedding, relation_embedding):
    hi = sample[:, 0]
    ri = sample[:, 1]
    ti = sample[:, 2]
    out = _transe_sc(hi, ri, ti, entity_embedding, relation_embedding)
    return out[:, None]
